# baseline (speedup 1.0000x reference)
import functools

import jax
import jax.numpy as jnp
from jax.experimental import pallas as pl
from jax.experimental.pallas import tpu as pltpu


_LOG2E = 1.4426950408889634


def _proj_kernel(x_ref, w_ref, a1_ref, a2_ref, h_ref, f1_ref, f2_ref, f2max_ref,
                 xsum_ref):
    x = x_ref[0]
    h = jnp.dot(x, w_ref[...], preferred_element_type=jnp.float32)
    h_ref[0] = h
    f1 = jnp.dot(h, a1_ref[...], preferred_element_type=jnp.float32) * _LOG2E
    f2 = jnp.dot(h, a2_ref[...], preferred_element_type=jnp.float32) * _LOG2E
    f1_ref[0] = f1
    f2_ref[0] = f2
    f2max_ref[0] = jnp.max(f2, axis=0, keepdims=True)
    xsum_ref[0] = jnp.sum(x, axis=0, keepdims=True)


def _gat_kernel(a_ref, h_ref, f1_ref, f2t_ref, f2max_ref, acc_ref):
    i = pl.program_id(1)
    f1 = f1_ref[0]
    tmax = f1 + f2max_ref[0]
    mrow = jnp.maximum(tmax, 0.2 * tmax)
    t = f1 + f2t_ref[0]
    e = jnp.maximum(t, 0.2 * t)
    q = jnp.exp2(e - mrow)
    p = jnp.where(a_ref[0] > 0.0, q, 0.0)
    s = jnp.sum(p, axis=1, keepdims=True)
    s = jnp.maximum(s, jnp.float32(1e-30))
    hp = jnp.dot(p.astype(jnp.bfloat16), h_ref[0],
                 preferred_element_type=jnp.float32) / s
    contrib = hp + 2.0 * jnp.maximum(hp, 0.0)
    part = jnp.sum(contrib, axis=0, keepdims=True)

    @pl.when(i == 0)
    def _():
        acc_ref[...] = jnp.zeros_like(acc_ref)

    acc_ref[...] += part[None]


def _mlp_kernel(inv_pool, acc_ref, xsum_ref, w1_ref, b1_ref, w2_ref, b2_ref, out_ref):
    xm = (acc_ref[...] + xsum_ref[...]) * inv_pool
    hmid = jnp.dot(xm, w1_ref[...], preferred_element_type=jnp.float32) + b1_ref[...]
    hmid = jnp.maximum(hmid, 0.0)
    out_ref[...] = jnp.dot(hmid, w2_ref[...], preferred_element_type=jnp.float32) + b2_ref[...]


def kernel(X, A, W_gat, a_gat, W1, b1, W2, b2):
    B, N, F = X.shape
    H = W1.shape[1]
    BLK = 512
    a1 = a_gat[:F]
    a2 = a_gat[F:]

    h, f1, f2, f2max, xsum = pl.pallas_call(
        _proj_kernel,
        grid=(B,),
        in_specs=[
            pl.BlockSpec((1, N, F), lambda b: (b, 0, 0)),
            pl.BlockSpec((F, F), lambda b: (0, 0)),
            pl.BlockSpec((F, 1), lambda b: (0, 0)),
            pl.BlockSpec((F, 1), lambda b: (0, 0)),
        ],
        out_specs=[
            pl.BlockSpec((1, N, F), lambda b: (b, 0, 0)),
            pl.BlockSpec((1, N, 1), lambda b: (b, 0, 0)),
            pl.BlockSpec((1, N, 1), lambda b: (b, 0, 0)),
            pl.BlockSpec((1, 1, 1), lambda b: (b, 0, 0)),
            pl.BlockSpec((1, 1, F), lambda b: (b, 0, 0)),
        ],
        out_shape=[
            jax.ShapeDtypeStruct((B, N, F), jnp.float32),
            jax.ShapeDtypeStruct((B, N, 1), jnp.float32),
            jax.ShapeDtypeStruct((B, N, 1), jnp.float32),
            jax.ShapeDtypeStruct((B, 1, 1), jnp.float32),
            jax.ShapeDtypeStruct((B, 1, F), jnp.float32),
        ],
    )(X, W_gat, a1, a2)

    f2t = jnp.reshape(f2, (B, 1, N))

    acc = pl.pallas_call(
        _gat_kernel,
        grid=(B, N // BLK),
        in_specs=[
            pl.BlockSpec((1, BLK, N), lambda b, i: (b, i, 0)),
            pl.BlockSpec((1, N, F), lambda b, i: (b, 0, 0)),
            pl.BlockSpec((1, BLK, 1), lambda b, i: (b, i, 0)),
            pl.BlockSpec((1, 1, N), lambda b, i: (b, 0, 0)),
            pl.BlockSpec((1, 1, 1), lambda b, i: (b, 0, 0)),
        ],
        out_specs=pl.BlockSpec((1, 1, F), lambda b, i: (b, 0, 0)),
        out_shape=jax.ShapeDtypeStruct((B, 1, F), jnp.float32),
        compiler_params=pltpu.CompilerParams(
            dimension_semantics=("parallel", "arbitrary"),
        ),
    )(A, h, f1, f2t, f2max)

    out = pl.pallas_call(
        functools.partial(_mlp_kernel, 1.0 / (4.0 * N)),
        in_specs=[
            pl.BlockSpec((B, F), lambda: (0, 0)),
            pl.BlockSpec((B, F), lambda: (0, 0)),
            pl.BlockSpec((F, H), lambda: (0, 0)),
            pl.BlockSpec((1, H), lambda: (0, 0)),
            pl.BlockSpec((H, 1), lambda: (0, 0)),
            pl.BlockSpec((1, 1), lambda: (0, 0)),
        ],
        out_specs=pl.BlockSpec((B, 1), lambda: (0, 0)),
        out_shape=jax.ShapeDtypeStruct((B, 1), jnp.float32),
    )(acc.reshape(B, F), xsum.reshape(B, F), W1, b1.reshape(1, H), W2,
      b2.reshape(1, 1))

    return out


# speedup vs baseline: 1.0913x; 1.0913x over previous
import functools

import jax
import jax.numpy as jnp
from jax.experimental import pallas as pl
from jax.experimental.pallas import tpu as pltpu

_LOG2E = 1.4426950408889634


def _gat_kernel(nb, x_ref, w_ref, a1_ref, a2_ref, a_ref, acc_ref,
                h_s, f1_s, f2t_s, m_s):
    i = pl.program_id(1)

    @pl.when(i == 0)
    def _proj():
        x = x_ref[0]
        h = jnp.dot(x, w_ref[...], preferred_element_type=jnp.float32)
        h_s[...] = h
        f1 = jnp.dot(h, a1_ref[...], preferred_element_type=jnp.float32) * _LOG2E
        f2 = jnp.dot(h, a2_ref[...], preferred_element_type=jnp.float32) * _LOG2E
        f1_s[...] = f1
        f2t_s[...] = jnp.reshape(f2, f2t_s.shape)
        m_s[...] = jnp.max(f2, axis=0, keepdims=True)
        acc_ref[...] = jnp.sum(x, axis=0, keepdims=True)[None]

    @pl.when(i > 0)
    def _attend():
        blk = a_ref.shape[1]
        r0 = (i - 1) * blk
        f1 = f1_s[pl.ds(r0, blk), :]
        tmax = f1 + m_s[...]
        mrow = jnp.maximum(tmax, 0.2 * tmax)
        t = f1 + f2t_s[...]
        e = jnp.maximum(t, 0.2 * t)
        q = jnp.exp2(e - mrow)
        p = jnp.where(a_ref[0] > 0.0, q, 0.0)
        s = jnp.sum(p, axis=1, keepdims=True)
        s = jnp.maximum(s, jnp.float32(1e-30))
        hp = jnp.dot(p.astype(jnp.bfloat16), h_s[...],
                     preferred_element_type=jnp.float32) / s
        contrib = hp + 2.0 * jnp.maximum(hp, 0.0)
        acc_ref[...] += jnp.sum(contrib, axis=0, keepdims=True)[None]


def _mlp_kernel(inv_pool, acc_ref, w1_ref, b1_ref, w2_ref, b2_ref, out_ref):
    xm = acc_ref[...] * inv_pool
    hmid = jnp.dot(xm, w1_ref[...], preferred_element_type=jnp.float32) + b1_ref[...]
    hmid = jnp.maximum(hmid, 0.0)
    out_ref[...] = jnp.dot(hmid, w2_ref[...], preferred_element_type=jnp.float32) + b2_ref[...]


def kernel(X, A, W_gat, a_gat, W1, b1, W2, b2):
    B, N, F = X.shape
    H = W1.shape[1]
    BLK = 512
    NB = N // BLK
    a1 = a_gat[:F]
    a2 = a_gat[F:]

    acc = pl.pallas_call(
        functools.partial(_gat_kernel, NB),
        grid=(B, NB + 1),
        in_specs=[
            pl.BlockSpec((1, N, F), lambda b, i: (b, 0, 0)),
            pl.BlockSpec((F, F), lambda b, i: (0, 0)),
            pl.BlockSpec((F, 1), lambda b, i: (0, 0)),
            pl.BlockSpec((F, 1), lambda b, i: (0, 0)),
            pl.BlockSpec((1, BLK, N), lambda b, i: (b, jnp.maximum(i, 1) - 1, 0)),
        ],
        out_specs=pl.BlockSpec((1, 1, F), lambda b, i: (b, 0, 0)),
        out_shape=jax.ShapeDtypeStruct((B, 1, F), jnp.float32),
        scratch_shapes=[
            pltpu.VMEM((N, F), jnp.float32),
            pltpu.VMEM((N, 1), jnp.float32),
            pltpu.VMEM((1, N), jnp.float32),
            pltpu.VMEM((1, 1), jnp.float32),
        ],
        compiler_params=pltpu.CompilerParams(
            dimension_semantics=("parallel", "arbitrary"),
        ),
    )(X, W_gat, a1, a2, A)

    out = pl.pallas_call(
        functools.partial(_mlp_kernel, 1.0 / (4.0 * N)),
        in_specs=[
            pl.BlockSpec((B, F), lambda: (0, 0)),
            pl.BlockSpec((F, H), lambda: (0, 0)),
            pl.BlockSpec((1, H), lambda: (0, 0)),
            pl.BlockSpec((H, 1), lambda: (0, 0)),
            pl.BlockSpec((1, 1), lambda: (0, 0)),
        ],
        out_specs=pl.BlockSpec((B, 1), lambda: (0, 0)),
        out_shape=jax.ShapeDtypeStruct((B, 1), jnp.float32),
    )(acc.reshape(B, F), W1, b1.reshape(1, H), W2, b2.reshape(1, 1))

    return out


# speedup vs baseline: 1.1949x; 1.0950x over previous
import functools

import jax
import jax.numpy as jnp
from jax.experimental import pallas as pl
from jax.experimental.pallas import tpu as pltpu

_LOG2E = 1.4426950408889634


def _gat_kernel(nb, x_ref, w_ref, a1_ref, a2_ref, a_ref, acc_ref,
                h_s, f1_s, f2t_s, m_s):
    i = pl.program_id(1)

    @pl.when(i == 0)
    def _proj():
        x = x_ref[0]
        h = jnp.dot(x, w_ref[...], preferred_element_type=jnp.float32)
        h_s[...] = h
        f1 = jnp.dot(h, a1_ref[...], preferred_element_type=jnp.float32) * _LOG2E
        f2 = jnp.dot(h, a2_ref[...], preferred_element_type=jnp.float32) * _LOG2E
        f1_s[...] = f1
        f2t_s[...] = jnp.reshape(f2, f2t_s.shape)
        m_s[...] = jnp.max(f2, axis=0, keepdims=True)
        acc_ref[...] = jnp.sum(x, axis=0, keepdims=True)[None]

    @pl.when(i > 0)
    def _attend():
        blk = a_ref.shape[1]
        r0 = (i - 1) * blk
        f1 = f1_s[pl.ds(r0, blk), :]
        tmax = f1 + m_s[...]
        mrow = jnp.maximum(tmax, 0.2 * tmax)
        t = f1 + f2t_s[...]
        e = jnp.maximum(t, 0.2 * t)
        q = jnp.exp2(e - mrow)
        p = jnp.where(a_ref[0] > 0.0, q, 0.0).astype(jnp.bfloat16)
        s = jnp.dot(p, jnp.ones((t.shape[1], 1), jnp.bfloat16),
                    preferred_element_type=jnp.float32)
        s = jnp.maximum(s, jnp.float32(1e-30))
        hp = jnp.dot(p, h_s[...], preferred_element_type=jnp.float32) / s
        contrib = hp + 2.0 * jnp.maximum(hp, 0.0)
        part = jnp.dot(jnp.ones((1, blk), jnp.float32), contrib,
                       preferred_element_type=jnp.float32)
        acc_ref[...] += part[None]


def _mlp_kernel(inv_pool, acc_ref, w1_ref, b1_ref, w2_ref, b2_ref, out_ref):
    xm = acc_ref[...] * inv_pool
    hmid = jnp.dot(xm, w1_ref[...], preferred_element_type=jnp.float32) + b1_ref[...]
    hmid = jnp.maximum(hmid, 0.0)
    out_ref[...] = jnp.dot(hmid, w2_ref[...], preferred_element_type=jnp.float32) + b2_ref[...]


def kernel(X, A, W_gat, a_gat, W1, b1, W2, b2):
    B, N, F = X.shape
    H = W1.shape[1]
    BLK = 512
    NB = N // BLK
    a1 = a_gat[:F]
    a2 = a_gat[F:]

    acc = pl.pallas_call(
        functools.partial(_gat_kernel, NB),
        grid=(B, NB + 1),
        in_specs=[
            pl.BlockSpec((1, N, F), lambda b, i: (b, 0, 0)),
            pl.BlockSpec((F, F), lambda b, i: (0, 0)),
            pl.BlockSpec((F, 1), lambda b, i: (0, 0)),
            pl.BlockSpec((F, 1), lambda b, i: (0, 0)),
            pl.BlockSpec((1, BLK, N), lambda b, i: (b, jnp.maximum(i, 1) - 1, 0)),
        ],
        out_specs=pl.BlockSpec((1, 1, F), lambda b, i: (b, 0, 0)),
        out_shape=jax.ShapeDtypeStruct((B, 1, F), jnp.float32),
        scratch_shapes=[
            pltpu.VMEM((N, F), jnp.float32),
            pltpu.VMEM((N, 1), jnp.float32),
            pltpu.VMEM((1, N), jnp.float32),
            pltpu.VMEM((1, 1), jnp.float32),
        ],
        compiler_params=pltpu.CompilerParams(
            dimension_semantics=("parallel", "arbitrary"),
        ),
    )(X, W_gat, a1, a2, A)

    out = pl.pallas_call(
        functools.partial(_mlp_kernel, 1.0 / (4.0 * N)),
        in_specs=[
            pl.BlockSpec((B, F), lambda: (0, 0)),
            pl.BlockSpec((F, H), lambda: (0, 0)),
            pl.BlockSpec((1, H), lambda: (0, 0)),
            pl.BlockSpec((H, 1), lambda: (0, 0)),
            pl.BlockSpec((1, 1), lambda: (0, 0)),
        ],
        out_specs=pl.BlockSpec((B, 1), lambda: (0, 0)),
        out_shape=jax.ShapeDtypeStruct((B, 1), jnp.float32),
    )(acc.reshape(B, F), W1, b1.reshape(1, H), W2, b2.reshape(1, 1))

    return out
